# Initial kernel scaffold; baseline (speedup 1.0000x reference)
import jax
import jax.numpy as jnp
from jax.experimental import pallas as pl


def kernel(x, W):
    raise NotImplementedError("write your pallas kernel here")


# speedup vs baseline: 1.8341x; 1.8341x over previous
import functools

import jax
import jax.numpy as jnp
from jax import lax
from jax.experimental import pallas as pl

DIM = 4096
N_EXPERTS = 64
TOPK = 8
N_GROUPS = 8
GROUP_SIZE = N_EXPERTS // N_GROUPS
TOPK_GROUPS = 4
ROUTE_SCALE = 2.5

BLK = 256


def _gate_body(x_ref, w_ref, wout_ref, iout_ref):
    x = x_ref[...].astype(jnp.bfloat16)
    w = w_ref[...].astype(jnp.bfloat16)
    logits = lax.dot_general(
        x, w, (((1,), (1,)), ((), ())),
        preferred_element_type=jnp.float32,
    )
    scores = jax.nn.sigmoid(logits)

    iota_e = lax.broadcasted_iota(jnp.int32, (BLK, N_EXPERTS), 1)
    gid = iota_e // GROUP_SIZE

    gmax = [
        jnp.max(scores[:, g * GROUP_SIZE:(g + 1) * GROUP_SIZE], axis=1,
                keepdims=True)
        for g in range(N_GROUPS)
    ]
    G = jnp.concatenate(gmax, axis=1)

    iota_g = lax.broadcasted_iota(jnp.int32, (BLK, N_GROUPS), 1)
    keep = jnp.zeros((BLK, N_EXPERTS), dtype=jnp.bool_)
    for _ in range(TOPK_GROUPS):
        m = jnp.max(G, axis=1, keepdims=True)
        sel = jnp.min(jnp.where(G == m, iota_g, N_GROUPS), axis=1,
                      keepdims=True)
        keep = keep | (gid == sel)
        G = jnp.where(iota_g == sel, -1.0, G)

    masked = jnp.where(keep, scores, 0.0)

    idx_cols = []
    val_cols = []
    for _ in range(TOPK):
        m = jnp.max(masked, axis=1, keepdims=True)
        sel = jnp.min(jnp.where(masked == m, iota_e, N_EXPERTS), axis=1,
                      keepdims=True)
        idx_cols.append(sel)
        val_cols.append(m)
        masked = jnp.where(iota_e == sel, -1.0, masked)

    indices = jnp.concatenate(idx_cols, axis=1)
    weights = jnp.concatenate(val_cols, axis=1)
    weights = weights * (ROUTE_SCALE / jnp.sum(weights, axis=1, keepdims=True))

    wout_ref[...] = weights
    iout_ref[...] = indices


@jax.jit
def kernel(x, W):
    n_tok = x.shape[0]
    grid = n_tok // BLK
    wout, iout = pl.pallas_call(
        _gate_body,
        grid=(grid,),
        in_specs=[
            pl.BlockSpec((BLK, DIM), lambda i: (i, 0)),
            pl.BlockSpec((N_EXPERTS, DIM), lambda i: (0, 0)),
        ],
        out_specs=[
            pl.BlockSpec((BLK, TOPK), lambda i: (i, 0)),
            pl.BlockSpec((BLK, TOPK), lambda i: (i, 0)),
        ],
        out_shape=[
            jax.ShapeDtypeStruct((n_tok, TOPK), jnp.float32),
            jax.ShapeDtypeStruct((n_tok, TOPK), jnp.int32),
        ],
    )(x, W)
    return wout, iout


# speedup vs baseline: 2.3239x; 1.2671x over previous
import functools

import jax
import jax.numpy as jnp
from jax import lax
from jax.experimental import pallas as pl
from jax.experimental.pallas import tpu as pltpu
from jax.experimental.pallas import tpu_sc as plsc

DIM = 4096
N_EXPERTS = 64
TOPK = 8
N_GROUPS = 8
GROUP_SIZE = N_EXPERTS // N_GROUPS
TOPK_GROUPS = 4
ROUTE_SCALE = 2.5

BLK = 256
NW = 32
GSEL_ROW = 8


def _scores_body(x_ref, w_ref, s_ref):
    x = x_ref[...].astype(jnp.bfloat16)
    w = w_ref[...].astype(jnp.bfloat16)
    logits = lax.dot_general(
        x, w, (((1,), (1,)), ((), ())),
        preferred_element_type=jnp.float32,
    )
    s_ref[...] = jax.nn.sigmoid(logits)


def _iota16():
    return lax.broadcasted_iota(jnp.int32, (16,), 0)


def _routing_body(tpw, scores_hbm, w_hbm, i_hbm, slab, gsel, wslab, islab):
    nc = 2
    wid = lax.axis_index("s") * nc + lax.axis_index("c")
    base = wid * tpw

    pltpu.sync_copy(scores_hbm.at[pl.ds(base * N_EXPERTS, tpw * N_EXPERTS)],
                    slab)

    iota = _iota16()
    lo8 = iota & 7
    half = iota >= 8

    def batch_body(b, carry):
        trow = (b * 16 + iota) * N_EXPERTS
        gmax = []
        for g in range(N_GROUPS):
            m = plsc.load_gather(slab, [trow + (g * GROUP_SIZE)])
            for j in range(1, GROUP_SIZE):
                m = jnp.maximum(
                    m, plsc.load_gather(slab, [trow + (g * GROUP_SIZE + j)]))
            gmax.append(m)
        for k in range(TOPK_GROUPS):
            mx = gmax[0]
            gi = jnp.zeros((16,), jnp.int32)
            for g in range(1, N_GROUPS):
                c = gmax[g] > mx
                mx = jnp.where(c, gmax[g], mx)
                gi = jnp.where(c, g, gi)
            plsc.store_scatter(gsel, [(b * 16 + iota) * GSEL_ROW + k], gi)
            for g in range(N_GROUPS):
                gmax[g] = jnp.where(gi == g, -1.0, gmax[g])
        return carry

    lax.fori_loop(0, tpw // 16, batch_body, 0)

    pat01 = half.astype(jnp.int32)
    pat23 = pat01 + 2
    in8 = iota < 8

    def tok_body(t, carry):
        ga = plsc.load_gather(gsel, [t * GSEL_ROW + pat01])
        gb = plsc.load_gather(gsel, [t * GSEL_ROW + pat23])
        expa = ga * GROUP_SIZE + lo8
        expb = gb * GROUP_SIZE + lo8
        va = plsc.load_gather(slab, [t * N_EXPERTS + expa])
        vb = plsc.load_gather(slab, [t * N_EXPERTS + expb])
        ska, sva = plsc.sort_key_val(va, expa, descending=True)
        skb, svb = plsc.sort_key_val(vb, expb, descending=True)
        rkb = lax.rev(skb, (0,))
        rvb = lax.rev(svb, (0,))
        c = ska >= rkb
        mk = jnp.where(c, ska, rkb)
        mv = jnp.where(c, sva, rvb)
        fk, fv = plsc.sort_key_val(mk, mv, descending=True)
        w8 = jnp.where(in8, fk, 0.0)
        s = lax.broadcast_in_dim(jnp.sum(w8), (16,), ())
        wout = w8 * ROUTE_SCALE / s
        plsc.store_scatter(wslab, [t * TOPK + lo8], wout, mask=in8)
        plsc.store_scatter(islab, [t * TOPK + lo8], fv, mask=in8)
        return carry

    lax.fori_loop(0, tpw, tok_body, 0)

    pltpu.sync_copy(wslab, w_hbm.at[pl.ds(base * TOPK, tpw * TOPK)])
    pltpu.sync_copy(islab, i_hbm.at[pl.ds(base * TOPK, tpw * TOPK)])


@jax.jit
def kernel(x, W):
    n_tok = x.shape[0]

    scores = pl.pallas_call(
        _scores_body,
        grid=(n_tok // BLK,),
        in_specs=[
            pl.BlockSpec((BLK, DIM), lambda i: (i, 0)),
            pl.BlockSpec((N_EXPERTS, DIM), lambda i: (0, 0)),
        ],
        out_specs=pl.BlockSpec((BLK, N_EXPERTS), lambda i: (i, 0)),
        out_shape=jax.ShapeDtypeStruct((n_tok, N_EXPERTS), jnp.float32),
    )(x, W)

    tpw = n_tok // NW
    mesh = plsc.VectorSubcoreMesh(core_axis_name="c", subcore_axis_name="s")
    w, i = pl.kernel(
        functools.partial(_routing_body, tpw),
        out_type=[
            jax.ShapeDtypeStruct((n_tok * TOPK,), jnp.float32),
            jax.ShapeDtypeStruct((n_tok * TOPK,), jnp.int32),
        ],
        mesh=mesh,
        compiler_params=pltpu.CompilerParams(needs_layout_passes=False),
        scratch_types=[
            pltpu.VMEM((tpw * N_EXPERTS,), jnp.float32),
            pltpu.VMEM((tpw * GSEL_ROW + 16,), jnp.int32),
            pltpu.VMEM((tpw * TOPK,), jnp.float32),
            pltpu.VMEM((tpw * TOPK,), jnp.int32),
        ],
    )(scores.reshape(-1))
    return w.reshape(n_tok, TOPK), i.reshape(n_tok, TOPK)


# speedup vs baseline: 2.5181x; 1.0836x over previous
import functools

import jax
import jax.numpy as jnp
from jax import lax
from jax.experimental import pallas as pl
from jax.experimental.pallas import tpu as pltpu
from jax.experimental.pallas import tpu_sc as plsc

DIM = 4096
N_EXPERTS = 64
TOPK = 8
N_GROUPS = 8
GROUP_SIZE = N_EXPERTS // N_GROUPS
TOPK_GROUPS = 4
ROUTE_SCALE = 2.5

BLK = 512
NW = 32
GSEL_ROW = 8


def _scores_body(x_ref, w_ref, s_ref):
    x = x_ref[...].astype(jnp.bfloat16)
    w = w_ref[...].astype(jnp.bfloat16)
    logits = lax.dot_general(
        x, w, (((1,), (1,)), ((), ())),
        preferred_element_type=jnp.float32,
    )
    s_ref[...] = jax.nn.sigmoid(logits)


def _iota16():
    return lax.broadcasted_iota(jnp.int32, (16,), 0)


def _routing_body(tpw, scores_hbm, w_hbm, i_hbm, slab, gsel, wslab, islab):
    nc = 2
    wid = lax.axis_index("s") * nc + lax.axis_index("c")
    base = wid * tpw

    pltpu.sync_copy(scores_hbm.at[pl.ds(base * N_EXPERTS, tpw * N_EXPERTS)],
                    slab)

    iota = _iota16()
    lo8 = iota & 7
    half = iota >= 8

    def batch_body(b, carry):
        trow = (b * 16 + iota) * N_EXPERTS
        gmax = []
        for g in range(N_GROUPS):
            m = plsc.load_gather(slab, [trow + (g * GROUP_SIZE)])
            for j in range(1, GROUP_SIZE):
                m = jnp.maximum(
                    m, plsc.load_gather(slab, [trow + (g * GROUP_SIZE + j)]))
            gmax.append(m)
        for k in range(TOPK_GROUPS):
            mx = gmax[0]
            gi = jnp.zeros((16,), jnp.int32)
            for g in range(1, N_GROUPS):
                c = gmax[g] > mx
                mx = jnp.where(c, gmax[g], mx)
                gi = jnp.where(c, g, gi)
            plsc.store_scatter(gsel, [(b * 16 + iota) * GSEL_ROW + k], gi)
            for g in range(N_GROUPS):
                gmax[g] = jnp.where(gi == g, -1.0, gmax[g])
        return carry

    lax.fori_loop(0, tpw // 16, batch_body, 0)

    pat01 = half.astype(jnp.int32)
    pat23 = pat01 + 2
    in8 = iota < 8

    def tok_body(t, carry):
        ga = plsc.load_gather(gsel, [t * GSEL_ROW + pat01])
        gb = plsc.load_gather(gsel, [t * GSEL_ROW + pat23])
        expa = ga * GROUP_SIZE + lo8
        expb = gb * GROUP_SIZE + lo8
        va = plsc.load_gather(slab, [t * N_EXPERTS + expa])
        vb = plsc.load_gather(slab, [t * N_EXPERTS + expb])
        ska, sva = plsc.sort_key_val(va, expa, descending=True)
        skb, svb = plsc.sort_key_val(vb, expb, descending=True)
        rkb = lax.rev(skb, (0,))
        rvb = lax.rev(svb, (0,))
        c = ska >= rkb
        mk = jnp.where(c, ska, rkb)
        mv = jnp.where(c, sva, rvb)
        fk, fv = plsc.sort_key_val(mk, mv, descending=True)
        w8 = jnp.where(in8, fk, 0.0)
        s = lax.broadcast_in_dim(jnp.sum(w8), (16,), ())
        wout = w8 * ROUTE_SCALE / s
        plsc.store_scatter(wslab, [t * TOPK + lo8], wout, mask=in8)
        plsc.store_scatter(islab, [t * TOPK + lo8], fv, mask=in8)
        return carry

    lax.fori_loop(0, tpw, tok_body, 0)

    pltpu.sync_copy(wslab, w_hbm.at[pl.ds(base * TOPK, tpw * TOPK)])
    pltpu.sync_copy(islab, i_hbm.at[pl.ds(base * TOPK, tpw * TOPK)])


@jax.jit
def kernel(x, W):
    n_tok = x.shape[0]

    scores = pl.pallas_call(
        _scores_body,
        grid=(n_tok // BLK,),
        in_specs=[
            pl.BlockSpec((BLK, DIM), lambda i: (i, 0)),
            pl.BlockSpec((N_EXPERTS, DIM), lambda i: (0, 0)),
        ],
        out_specs=pl.BlockSpec((BLK, N_EXPERTS), lambda i: (i, 0)),
        out_shape=jax.ShapeDtypeStruct((n_tok, N_EXPERTS), jnp.float32),
    )(x, W)

    tpw = n_tok // NW
    mesh = plsc.VectorSubcoreMesh(core_axis_name="c", subcore_axis_name="s")
    w, i = pl.kernel(
        functools.partial(_routing_body, tpw),
        out_type=[
            jax.ShapeDtypeStruct((n_tok * TOPK,), jnp.float32),
            jax.ShapeDtypeStruct((n_tok * TOPK,), jnp.int32),
        ],
        mesh=mesh,
        compiler_params=pltpu.CompilerParams(needs_layout_passes=False),
        scratch_types=[
            pltpu.VMEM((tpw * N_EXPERTS,), jnp.float32),
            pltpu.VMEM((tpw * GSEL_ROW + 16,), jnp.int32),
            pltpu.VMEM((tpw * TOPK,), jnp.float32),
            pltpu.VMEM((tpw * TOPK,), jnp.int32),
        ],
    )(scores.reshape(-1))
    return w.reshape(n_tok, TOPK), i.reshape(n_tok, TOPK)
